# baseline (speedup 1.0000x reference)
import jax
import jax.numpy as jnp
from jax.experimental import pallas as pl
from jax.experimental.pallas import tpu as pltpu

S, H, I = 2048, 768, 384
E = 7
EP = 128
NEG = -1e30


def _mm(a, b):
    return jax.lax.dot(a, b, preferred_element_type=jnp.float32)


def _dense_body(xr, xbr, wrr, rbr, wgsr, wusr, wdsr, wgr, wur, wdr, outr, wfull):
    e = pl.program_id(0)
    xb = xbr[...]

    @pl.when(e == 0)
    def _():
        h = jax.nn.silu(_mm(xb, wgsr[...])) * _mm(xb, wusr[...])
        outr[...] = _mm(h.astype(jnp.bfloat16), wdsr[...])
        probs = jax.nn.sigmoid(xr[...] @ wrr[...] + rbr[...])
        lane = jax.lax.broadcasted_iota(jnp.int32, (S, EP), 1)
        m0 = jnp.max(probs, axis=1, keepdims=True)
        i0 = jnp.min(jnp.where(probs == m0, lane, EP), axis=1, keepdims=True)
        probs1 = jnp.where(lane == i0, NEG, probs)
        m1 = jnp.max(probs1, axis=1, keepdims=True)
        i1 = jnp.min(jnp.where(probs1 == m1, lane, EP), axis=1, keepdims=True)
        wfull[...] = m0 * (lane == i0) + m1 * (lane == i1)

    onehot = (jax.lax.broadcasted_iota(jnp.int32, (EP, 1), 0) == e).astype(jnp.float32)
    w_e = wfull[...] @ onehot
    h = jax.nn.silu(_mm(xb, wgr[0])) * _mm(xb, wur[0])
    outr[...] += _mm(h.astype(jnp.bfloat16), wdr[0]) * w_e


def kernel(x, Wg_s, Wu_s, Wd_s, Wg, Wu, Wd, Wr, rbias):
    xf = x.reshape(S, H)
    xb = xf.astype(jnp.bfloat16)
    bf = jnp.bfloat16
    Wrp = jnp.zeros((H, EP), jnp.float32).at[:, :E].set(Wr)
    rbp = jnp.full((1, EP), NEG, jnp.float32).at[0, :E].set(rbias)

    out = pl.pallas_call(
        _dense_body,
        grid=(E,),
        in_specs=[
            pl.BlockSpec((S, H), lambda e: (0, 0)),
            pl.BlockSpec((S, H), lambda e: (0, 0)),
            pl.BlockSpec((H, EP), lambda e: (0, 0)),
            pl.BlockSpec((1, EP), lambda e: (0, 0)),
            pl.BlockSpec((H, I), lambda e: (0, 0)),
            pl.BlockSpec((H, I), lambda e: (0, 0)),
            pl.BlockSpec((I, H), lambda e: (0, 0)),
            pl.BlockSpec((1, H, I), lambda e: (e, 0, 0)),
            pl.BlockSpec((1, H, I), lambda e: (e, 0, 0)),
            pl.BlockSpec((1, I, H), lambda e: (e, 0, 0)),
        ],
        out_specs=pl.BlockSpec((S, H), lambda e: (0, 0)),
        out_shape=jax.ShapeDtypeStruct((S, H), jnp.float32),
        scratch_shapes=[pltpu.VMEM((S, EP), jnp.float32)],
        compiler_params=pltpu.CompilerParams(
            dimension_semantics=("arbitrary",),
        ),
    )(xf, xb, Wrp, rbp, Wg_s.astype(bf), Wu_s.astype(bf), Wd_s.astype(bf),
      Wg.astype(bf), Wu.astype(bf), Wd.astype(bf))
    return out.reshape(1, S, H)


# speedup vs baseline: 1.3822x; 1.3822x over previous
import jax
import jax.numpy as jnp
from jax.experimental import pallas as pl
from jax.experimental.pallas import tpu as pltpu

S, H, I = 2048, 768, 384
E = 7
EP = 128
NEG = -1e30


def _mm(a, b):
    return jax.lax.dot(a, b, preferred_element_type=jnp.float32)


def _dense_body(xr, wrr, rbr, wgsr, wusr, wdsr, wgr, wur, wdr, outr, wfull):
    e = pl.program_id(0)
    bf = jnp.bfloat16
    xb = xr[...].astype(bf)

    @pl.when(e == 0)
    def _():
        h = jax.nn.silu(_mm(xb, wgsr[...].astype(bf))) * _mm(xb, wusr[...].astype(bf))
        outr[...] = _mm(h.astype(bf), wdsr[...].astype(bf))
        probs = jax.nn.sigmoid(xr[...] @ wrr[...] + rbr[...])
        lane = jax.lax.broadcasted_iota(jnp.int32, (S, EP), 1)
        m0 = jnp.max(probs, axis=1, keepdims=True)
        i0 = jnp.min(jnp.where(probs == m0, lane, EP), axis=1, keepdims=True)
        probs1 = jnp.where(lane == i0, NEG, probs)
        m1 = jnp.max(probs1, axis=1, keepdims=True)
        i1 = jnp.min(jnp.where(probs1 == m1, lane, EP), axis=1, keepdims=True)
        wfull[...] = m0 * (lane == i0) + m1 * (lane == i1)

    onehot = (jax.lax.broadcasted_iota(jnp.int32, (EP, 1), 0) == e).astype(jnp.float32)
    w_e = wfull[...] @ onehot
    h = jax.nn.silu(_mm(xb, wgr[0].astype(bf))) * _mm(xb, wur[0].astype(bf))
    outr[...] += _mm(h.astype(bf), wdr[0].astype(bf)) * w_e


def kernel(x, Wg_s, Wu_s, Wd_s, Wg, Wu, Wd, Wr, rbias):
    xf = x.reshape(S, H)
    Wrp = jnp.zeros((H, EP), jnp.float32).at[:, :E].set(Wr)
    rbp = jnp.full((1, EP), NEG, jnp.float32).at[0, :E].set(rbias)

    out = pl.pallas_call(
        _dense_body,
        grid=(E,),
        in_specs=[
            pl.BlockSpec((S, H), lambda e: (0, 0)),
            pl.BlockSpec((H, EP), lambda e: (0, 0)),
            pl.BlockSpec((1, EP), lambda e: (0, 0)),
            pl.BlockSpec((H, I), lambda e: (0, 0)),
            pl.BlockSpec((H, I), lambda e: (0, 0)),
            pl.BlockSpec((I, H), lambda e: (0, 0)),
            pl.BlockSpec((1, H, I), lambda e: (e, 0, 0)),
            pl.BlockSpec((1, H, I), lambda e: (e, 0, 0)),
            pl.BlockSpec((1, I, H), lambda e: (e, 0, 0)),
        ],
        out_specs=pl.BlockSpec((S, H), lambda e: (0, 0)),
        out_shape=jax.ShapeDtypeStruct((S, H), jnp.float32),
        scratch_shapes=[pltpu.VMEM((S, EP), jnp.float32)],
        compiler_params=pltpu.CompilerParams(
            dimension_semantics=("arbitrary",),
        ),
    )(xf, Wrp, rbp, Wg_s, Wu_s, Wd_s, Wg, Wu, Wd)
    return out.reshape(1, S, H)
